# Initial kernel scaffold; baseline (speedup 1.0000x reference)
import jax
import jax.numpy as jnp
from jax.experimental import pallas as pl


def kernel(slab, img, lab, roi, mu, sigma, noise):
    raise NotImplementedError("write your pallas kernel here")


# speedup vs baseline: 1.6094x; 1.6094x over previous
import functools

import jax
import jax.numpy as jnp
from jax import lax
from jax.experimental import pallas as pl
from jax.experimental.pallas import tpu as pltpu
from jax.experimental.pallas import tpu_sc as plsc

D = H = W = 128
N = D * H * W
NC, NS = 2, 16
NW = NC * NS
PER_W = N // NW
CHUNK = 8192
NCHUNK = PER_W // CHUNK
LANES = 16


def _remap(s):
    five = jnp.full((LANES,), 5, jnp.int32)
    zero = jnp.zeros((LANES,), jnp.int32)
    return jnp.where(s < 5, s, jnp.where(s == 18, five, zero))


def _sc_kernel(slab_hbm, lab_hbm, noise_hbm, tab_hbm,
               simg_hbm, so_hbm, lo_hbm,
               slab_v, lab_v, noise_v, simg_v, so_v, lo_v, tab_v):
    wid = lax.axis_index("s") * NC + lax.axis_index("c")
    pltpu.sync_copy(tab_hbm, tab_v)

    def do_chunk(ci, _):
        base = wid * PER_W + ci * CHUNK
        pltpu.sync_copy(slab_hbm.at[pl.ds(base, CHUNK)], slab_v)
        pltpu.sync_copy(lab_hbm.at[pl.ds(base, CHUNK)], lab_v)
        pltpu.sync_copy(noise_hbm.at[pl.ds(base, CHUNK)], noise_v)

        def body(i, _):
            off = i * LANES
            s = slab_v[pl.ds(off, LANES)]
            mu_v = plsc.load_gather(tab_v, [s])
            sg_v = plsc.load_gather(tab_v, [s + 32])
            nz = noise_v[pl.ds(off, LANES)]
            simg_v[pl.ds(off, LANES)] = mu_v + sg_v * nz
            so_v[pl.ds(off, LANES)] = _remap(s)
            lo_v[pl.ds(off, LANES)] = _remap(lab_v[pl.ds(off, LANES)])
            return 0

        lax.fori_loop(0, CHUNK // LANES, body, 0)

        pltpu.sync_copy(simg_v, simg_hbm.at[pl.ds(base, CHUNK)])
        pltpu.sync_copy(so_v, so_hbm.at[pl.ds(base, CHUNK)])
        pltpu.sync_copy(lo_v, lo_hbm.at[pl.ds(base, CHUNK)])
        return 0

    lax.fori_loop(0, NCHUNK, do_chunk, 0)


@jax.jit
def _run(slab_f, lab_f, noise_f, tab):
    mesh = plsc.VectorSubcoreMesh(core_axis_name="c", subcore_axis_name="s")
    k = functools.partial(
        pl.kernel, mesh=mesh,
        compiler_params=pltpu.CompilerParams(needs_layout_passes=False),
        out_type=(
            jax.ShapeDtypeStruct((N,), jnp.float32),
            jax.ShapeDtypeStruct((N,), jnp.int32),
            jax.ShapeDtypeStruct((N,), jnp.int32),
        ),
        scratch_types=[
            pltpu.VMEM((CHUNK,), jnp.int32),
            pltpu.VMEM((CHUNK,), jnp.int32),
            pltpu.VMEM((CHUNK,), jnp.float32),
            pltpu.VMEM((CHUNK,), jnp.float32),
            pltpu.VMEM((CHUNK,), jnp.int32),
            pltpu.VMEM((CHUNK,), jnp.int32),
            pltpu.VMEM((64,), jnp.float32),
        ],
    )(_sc_kernel)
    return k(slab_f, lab_f, noise_f, tab)


def kernel(slab, img, lab, roi, mu, sigma, noise):
    slab_f = slab.reshape(N).astype(jnp.int32)
    lab_f = lab.reshape(N).astype(jnp.int32)
    noise_f = noise.reshape(N).astype(jnp.float32)
    tab = jnp.concatenate([
        jnp.pad(mu.astype(jnp.float32), (0, 32 - mu.shape[0])),
        jnp.pad(sigma.astype(jnp.float32), (0, 32 - sigma.shape[0])),
    ])
    simg_f, so_f, lo_f = _run(slab_f, lab_f, noise_f, tab)
    simg = simg_f.reshape(1, D, H, W)
    slab_out = so_f.reshape(1, D, H, W).astype(slab.dtype)
    rlab_out = lo_f.reshape(1, D, H, W).astype(lab.dtype)
    return (simg, slab_out, img.astype(jnp.float32), rlab_out, roi)


# speedup vs baseline: 2.3751x; 1.4758x over previous
import functools

import jax
import jax.numpy as jnp
from jax import lax
from jax.experimental import pallas as pl
from jax.experimental.pallas import tpu as pltpu
from jax.experimental.pallas import tpu_sc as plsc

D = H = W = 128
N = D * H * W
NC, NS = 2, 16
NW = NC * NS
PER_W = N // NW
CHUNK = 8192
NCHUNK = PER_W // CHUNK
LANES = 16


def _remap(s):
    five = jnp.full((LANES,), 5, jnp.int32)
    zero = jnp.zeros((LANES,), jnp.int32)
    return jnp.where(s < 5, s, jnp.where(s == 18, five, zero))


def _sc_kernel(slab_hbm, lab_hbm, noise_hbm, tab_hbm,
               simg_hbm, so_hbm, lo_hbm,
               slab_v, lab_v, noise_v, simg_v, so_v, lo_v, tab_v,
               sem_i0, sem_i1, sem_o0, sem_o1):
    wid = lax.axis_index("s") * NC + lax.axis_index("c")
    sem_in = (sem_i0, sem_i1)
    sem_out = (sem_o0, sem_o1)
    pltpu.sync_copy(tab_hbm, tab_v)

    def start_in(ci, slot):
        base = wid * PER_W + ci * CHUNK
        return (
            pltpu.async_copy(slab_hbm.at[pl.ds(base, CHUNK)],
                             slab_v.at[slot], sem_in[slot]),
            pltpu.async_copy(lab_hbm.at[pl.ds(base, CHUNK)],
                             lab_v.at[slot], sem_in[slot]),
            pltpu.async_copy(noise_hbm.at[pl.ds(base, CHUNK)],
                             noise_v.at[slot], sem_in[slot]),
        )

    def start_out(ci, slot):
        base = wid * PER_W + ci * CHUNK
        return (
            pltpu.async_copy(simg_v.at[slot],
                             simg_hbm.at[pl.ds(base, CHUNK)], sem_out[slot]),
            pltpu.async_copy(so_v.at[slot],
                             so_hbm.at[pl.ds(base, CHUNK)], sem_out[slot]),
            pltpu.async_copy(lo_v.at[slot],
                             lo_hbm.at[pl.ds(base, CHUNK)], sem_out[slot]),
        )

    def compute(slot):
        @plsc.parallel_loop(0, CHUNK // LANES, unroll=4)
        def _(i):
            off = i * LANES
            s = slab_v[slot, pl.ds(off, LANES)]
            mu_v = plsc.load_gather(tab_v, [s])
            sg_v = plsc.load_gather(tab_v, [s + 32])
            nz = noise_v[slot, pl.ds(off, LANES)]
            simg_v[slot, pl.ds(off, LANES)] = mu_v + sg_v * nz
            so_v[slot, pl.ds(off, LANES)] = _remap(s)
            lo_v[slot, pl.ds(off, LANES)] = _remap(lab_v[slot, pl.ds(off, LANES)])

    handles_in = [None, None]
    handles_out = [None, None]
    handles_in[0] = start_in(0, 0)
    for ci in range(NCHUNK):
        slot = ci & 1
        if ci + 1 < NCHUNK:
            handles_in[1 - slot] = start_in(ci + 1, 1 - slot)
        for h in handles_in[slot]:
            h.wait()
        if handles_out[slot] is not None:
            for h in handles_out[slot]:
                h.wait()
        compute(slot)
        handles_out[slot] = start_out(ci, slot)
    for slot in (0, 1):
        if handles_out[slot] is not None:
            for h in handles_out[slot]:
                h.wait()


@jax.jit
def _run(slab_f, lab_f, noise_f, tab):
    mesh = plsc.VectorSubcoreMesh(core_axis_name="c", subcore_axis_name="s")
    k = functools.partial(
        pl.kernel, mesh=mesh,
        compiler_params=pltpu.CompilerParams(needs_layout_passes=False),
        out_type=(
            jax.ShapeDtypeStruct((N,), jnp.float32),
            jax.ShapeDtypeStruct((N,), jnp.int32),
            jax.ShapeDtypeStruct((N,), jnp.int32),
        ),
        scratch_types=[
            pltpu.VMEM((2, CHUNK), jnp.int32),
            pltpu.VMEM((2, CHUNK), jnp.int32),
            pltpu.VMEM((2, CHUNK), jnp.float32),
            pltpu.VMEM((2, CHUNK), jnp.float32),
            pltpu.VMEM((2, CHUNK), jnp.int32),
            pltpu.VMEM((2, CHUNK), jnp.int32),
            pltpu.VMEM((64,), jnp.float32),
            pltpu.SemaphoreType.DMA,
            pltpu.SemaphoreType.DMA,
            pltpu.SemaphoreType.DMA,
            pltpu.SemaphoreType.DMA,
        ],
    )(_sc_kernel)
    return k(slab_f, lab_f, noise_f, tab)


def kernel(slab, img, lab, roi, mu, sigma, noise):
    slab_f = slab.reshape(N).astype(jnp.int32)
    lab_f = lab.reshape(N).astype(jnp.int32)
    noise_f = noise.reshape(N).astype(jnp.float32)
    tab = jnp.concatenate([
        jnp.pad(mu.astype(jnp.float32), (0, 32 - mu.shape[0])),
        jnp.pad(sigma.astype(jnp.float32), (0, 32 - sigma.shape[0])),
    ])
    simg_f, so_f, lo_f = _run(slab_f, lab_f, noise_f, tab)
    simg = simg_f.reshape(1, D, H, W)
    slab_out = so_f.reshape(1, D, H, W).astype(slab.dtype)
    rlab_out = lo_f.reshape(1, D, H, W).astype(lab.dtype)
    return (simg, slab_out, img.astype(jnp.float32), rlab_out, roi)
